# baseline (speedup 1.0000x reference)
import functools

import jax
import jax.numpy as jnp
from jax import lax
from jax.experimental import pallas as pl
from jax.experimental.pallas import tpu as pltpu
from jax.experimental.pallas import tpu_sc as plsc

N = 10000
E = 320000
H = 32
BN_EPS = 1e-5

NC = 2
NS = 16
NT = NC * NS
CHUNK = 128
G = 2
K = 80
E_PAD = NT * K * CHUNK
ROWS_PER_TILE = 632
NROWS = NS * ROWS_PER_TILE


def _sc_segment_sum(y, src3, dst3, zeros):
    y = jnp.concatenate(
        [y, jnp.zeros((NROWS - N, H), jnp.float32)], axis=0)
    mesh = plsc.VectorSubcoreMesh(core_axis_name="c", subcore_axis_name="s")

    @functools.partial(
        pl.kernel,
        out_type=jax.ShapeDtypeStruct((NC, NROWS, H), jnp.float32),
        mesh=mesh,
        compiler_params=pltpu.CompilerParams(use_tc_tiling_on_sc=False, skip_device_barrier=True),
        scratch_types=[
            pltpu.VMEM((K, CHUNK), jnp.int32),
            pltpu.VMEM((K, CHUNK), jnp.int32),
            pltpu.VMEM((2 * G, CHUNK, H), jnp.float32),
            pltpu.VMEM_SHARED((NROWS, H), jnp.float32),
            pltpu.VMEM_SHARED((NROWS, H), jnp.float32),
            pltpu.SemaphoreType.DMA,
            pltpu.SemaphoreType.DMA,
        ],
    )
    def k(y_hbm, src_hbm, dst_hbm, zeros_hbm, out_hbm, idx_s, idx_d, rows,
          y_sh, agg_sh, gsem, ssem):
        cid = lax.axis_index("c")
        sid = lax.axis_index("s")
        wid = sid * NC + cid
        row0 = sid * ROWS_PER_TILE
        pltpu.sync_copy(src_hbm.at[wid], idx_s)
        pltpu.sync_copy(dst_hbm.at[wid], idx_d)
        pltpu.sync_copy(y_hbm.at[pl.ds(row0, ROWS_PER_TILE)],
                        y_sh.at[pl.ds(row0, ROWS_PER_TILE)])
        pltpu.sync_copy(zeros_hbm.at[pl.ds(row0, ROWS_PER_TILE)],
                        agg_sh.at[pl.ds(row0, ROWS_PER_TILE)])
        plsc.subcore_barrier()

        def fire(base, half):
            return [pltpu.async_copy(y_sh.at[idx_s.at[base + b]],
                                     rows.at[half * G + b], gsem)
                    for b in range(G)]

        def scatter(base, half, gds):
            sds = []
            for b in range(G):
                gds[b].wait()
                sds.append(pltpu.async_copy(rows.at[half * G + b],
                                            agg_sh.at[idx_d.at[base + b]],
                                            ssem, add=True))
            return sds

        def body(t, carry):
            base = t * (2 * G)
            gA = fire(base, 0)
            sA = scatter(base, 0, gA)
            gB = fire(base + G, 1)
            for d in sA:
                d.wait()
            sB = scatter(base + G, 1, gB)
            for d in sB:
                d.wait()
            return carry

        lax.fori_loop(0, K // (2 * G), body, 0)
        plsc.subcore_barrier()
        pltpu.sync_copy(agg_sh.at[pl.ds(row0, ROWS_PER_TILE)],
                        out_hbm.at[cid, pl.ds(row0, ROWS_PER_TILE)])

    return k(y, src3, dst3, zeros)


_BR = 1000


def _tc_matmul(x, w):
    a, b = x.shape[1], w.shape[1]

    def body(x_ref, w_ref, o_ref):
        o_ref[...] = jnp.dot(x_ref[...], w_ref[...],
                             preferred_element_type=jnp.float32)

    return pl.pallas_call(
        body,
        grid=(N // _BR,),
        in_specs=[pl.BlockSpec((_BR, a), lambda i: (i, 0)),
                  pl.BlockSpec((a, b), lambda i: (0, 0))],
        out_specs=pl.BlockSpec((_BR, b), lambda i: (i, 0)),
        out_shape=jax.ShapeDtypeStruct((N, b), jnp.float32),
    )(x, w)


def _tc_mid(y, aggs, b1s, w2, b2, w1n):
    c = w2.shape[1]
    bn = w1n.shape[1]

    def body(y_ref, a0_ref, a1_ref, b1_ref, w2_ref, b2_ref, w1_ref, o_ref):
        u = jnp.maximum(y_ref[...] + a0_ref[0] + a1_ref[0] + b1_ref[...], 0.0)
        h = jnp.maximum(
            jnp.dot(u, w2_ref[...], preferred_element_type=jnp.float32)
            + b2_ref[...], 0.0)
        o_ref[...] = jnp.dot(h, w1_ref[...], preferred_element_type=jnp.float32)

    return pl.pallas_call(
        body,
        grid=(N // _BR,),
        in_specs=[
            pl.BlockSpec((_BR, H), lambda i: (i, 0)),
            pl.BlockSpec((1, _BR, H), lambda i: (0, i, 0)),
            pl.BlockSpec((1, _BR, H), lambda i: (1, i, 0)),
            pl.BlockSpec((1, H), lambda i: (0, 0)),
            pl.BlockSpec((H, c), lambda i: (0, 0)),
            pl.BlockSpec((1, c), lambda i: (0, 0)),
            pl.BlockSpec((c, bn), lambda i: (0, 0)),
        ],
        out_specs=pl.BlockSpec((_BR, bn), lambda i: (i, 0)),
        out_shape=jax.ShapeDtypeStruct((N, bn), jnp.float32),
    )(y, aggs, aggs, b1s, w2, b2, w1n)


def _tc_final(y, aggs, b1s, w2, b2):
    c = w2.shape[1]
    nb = N // _BR

    def body(y_ref, a0_ref, a1_ref, b1_ref, w2_ref, b2_ref, o_ref):
        i = pl.program_id(0)
        u = jnp.maximum(y_ref[...] + a0_ref[0] + a1_ref[0] + b1_ref[...], 0.0)
        h = jnp.maximum(
            jnp.dot(u, w2_ref[...], preferred_element_type=jnp.float32)
            + b2_ref[...], 0.0)
        part = jnp.sum(h, axis=0, keepdims=True)

        @pl.when(i == 0)
        def _():
            o_ref[...] = part

        @pl.when(i > 0)
        def _():
            o_ref[...] += part

        @pl.when(i == nb - 1)
        def _():
            p = o_ref[...]
            m = jnp.max(p, axis=1, keepdims=True)
            e = jnp.exp(p - m)
            o_ref[...] = e / jnp.sum(e, axis=1, keepdims=True)

    return pl.pallas_call(
        body,
        grid=(nb,),
        in_specs=[
            pl.BlockSpec((_BR, H), lambda i: (i, 0)),
            pl.BlockSpec((1, _BR, H), lambda i: (0, i, 0)),
            pl.BlockSpec((1, _BR, H), lambda i: (1, i, 0)),
            pl.BlockSpec((1, H), lambda i: (0, 0)),
            pl.BlockSpec((H, c), lambda i: (0, 0)),
            pl.BlockSpec((1, c), lambda i: (0, 0)),
        ],
        out_specs=pl.BlockSpec((1, c), lambda i: (0, 0)),
        out_shape=jax.ShapeDtypeStruct((1, c), jnp.float32),
    )(y, aggs, aggs, b1s, w2, b2)


def kernel(x, edge_index, params):
    folded = []
    for p in params:
        s = p["g1"] / jnp.sqrt(1.0 + BN_EPS)
        w1s = p["W1"] * s[None, :]
        b1s = (p["b1"] * s + p["be1"]).reshape(1, -1)
        folded.append((w1s, b1s, p["W2"], p["b2"].reshape(1, -1)))

    pad = E_PAD - E
    src = jnp.concatenate(
        [edge_index[0], jnp.zeros((pad,), jnp.int32)]).reshape(NT, K, CHUNK)
    dst = jnp.concatenate(
        [edge_index[1], jnp.full((pad,), N, jnp.int32)]).reshape(NT, K, CHUNK)
    zeros = jnp.zeros((NROWS, H), jnp.float32)

    y = _tc_matmul(x, folded[0][0])
    for l in range(5):
        aggs = _sc_segment_sum(y, src, dst, zeros)
        w1s, b1s, w2, b2 = folded[l]
        if l < 4:
            y = _tc_mid(y, aggs, b1s, w2, b2, folded[l + 1][0])
        else:
            out = _tc_final(y, aggs, b1s, w2, b2)
    return out


# speedup vs baseline: 1.1463x; 1.1463x over previous
import functools

import jax
import jax.numpy as jnp
from jax import lax
from jax.experimental import pallas as pl
from jax.experimental.pallas import tpu as pltpu
from jax.experimental.pallas import tpu_sc as plsc

N = 10000
E = 320000
H = 32
BN_EPS = 1e-5

NC = 2
NS = 16
NT = NC * NS
CHUNK = 128
G = 8
K = 80
E_PAD = NT * K * CHUNK
ROWS_PER_TILE = 632
NROWS = NS * ROWS_PER_TILE


def _sc_segment_sum(y, src3, dst3, zeros):
    mesh = plsc.VectorSubcoreMesh(core_axis_name="c", subcore_axis_name="s")

    @functools.partial(
        pl.kernel,
        out_type=jax.ShapeDtypeStruct((NC, NROWS, H), jnp.float32),
        mesh=mesh,
        compiler_params=pltpu.CompilerParams(use_tc_tiling_on_sc=False, skip_device_barrier=True),
        scratch_types=[
            pltpu.VMEM((K, CHUNK), jnp.int32),
            pltpu.VMEM((K, CHUNK), jnp.int32),
            pltpu.VMEM((2 * G, CHUNK, H), jnp.float32),
            pltpu.VMEM_SHARED((NROWS, H), jnp.float32),
            pltpu.VMEM_SHARED((NROWS, H), jnp.float32),
            pltpu.SemaphoreType.DMA,
            pltpu.SemaphoreType.DMA,
        ],
    )
    def k(y_hbm, src_hbm, dst_hbm, zeros_hbm, out_hbm, idx_s, idx_d, rows,
          y_sh, agg_sh, gsem, ssem):
        cid = lax.axis_index("c")
        sid = lax.axis_index("s")
        wid = sid * NC + cid
        row0 = sid * ROWS_PER_TILE
        pltpu.sync_copy(src_hbm.at[wid], idx_s)
        pltpu.sync_copy(dst_hbm.at[wid], idx_d)
        pltpu.sync_copy(y_hbm.at[pl.ds(row0, ROWS_PER_TILE)],
                        y_sh.at[pl.ds(row0, ROWS_PER_TILE)])
        pltpu.sync_copy(zeros_hbm.at[pl.ds(row0, ROWS_PER_TILE)],
                        agg_sh.at[pl.ds(row0, ROWS_PER_TILE)])
        plsc.subcore_barrier()

        def fire(base, half):
            return [pltpu.async_copy(y_sh.at[idx_s.at[base + b]],
                                     rows.at[half * G + b], gsem)
                    for b in range(G)]

        def scatter(base, half, gds):
            sds = []
            for b in range(G):
                gds[b].wait()
                sds.append(pltpu.async_copy(rows.at[half * G + b],
                                            agg_sh.at[idx_d.at[base + b]],
                                            ssem, add=True))
            return sds

        def body(t, carry):
            base = t * (2 * G)
            gA = fire(base, 0)
            sA = scatter(base, 0, gA)
            gB = fire(base + G, 1)
            for d in sA:
                d.wait()
            sB = scatter(base + G, 1, gB)
            for d in sB:
                d.wait()
            return carry

        lax.fori_loop(0, K // (2 * G), body, 0)
        plsc.subcore_barrier()
        pltpu.sync_copy(agg_sh.at[pl.ds(row0, ROWS_PER_TILE)],
                        out_hbm.at[cid, pl.ds(row0, ROWS_PER_TILE)])

    return k(y, src3, dst3, zeros)


_BR = 1000
_BRM = 2528


def _tc_matmul(x, w):
    a, b = x.shape[1], w.shape[1]

    def body(x_ref, w_ref, o_ref):
        o_ref[...] = jnp.dot(x_ref[...], w_ref[...],
                             preferred_element_type=jnp.float32)

    return pl.pallas_call(
        body,
        grid=(NROWS // _BR,),
        in_specs=[pl.BlockSpec((_BR, a), lambda i: (i, 0)),
                  pl.BlockSpec((a, b), lambda i: (0, 0))],
        out_specs=pl.BlockSpec((_BR, b), lambda i: (i, 0)),
        out_shape=jax.ShapeDtypeStruct((NROWS, b), jnp.float32),
    )(x, w)


def _tc_mid(y, aggs, b1s, w2, b2, w1n):
    c = w2.shape[1]
    bn = w1n.shape[1]

    def body(y_ref, a0_ref, a1_ref, b1_ref, w2_ref, b2_ref, w1_ref, o_ref):
        u = jnp.maximum(y_ref[...] + a0_ref[0] + a1_ref[0] + b1_ref[...], 0.0)
        h = jnp.maximum(
            jnp.dot(u, w2_ref[...], preferred_element_type=jnp.float32)
            + b2_ref[...], 0.0)
        o_ref[...] = jnp.dot(h, w1_ref[...], preferred_element_type=jnp.float32)

    return pl.pallas_call(
        body,
        grid=(NROWS // _BRM,),
        in_specs=[
            pl.BlockSpec((_BRM, H), lambda i: (i, 0)),
            pl.BlockSpec((1, _BRM, H), lambda i: (0, i, 0)),
            pl.BlockSpec((1, _BRM, H), lambda i: (1, i, 0)),
            pl.BlockSpec((1, H), lambda i: (0, 0)),
            pl.BlockSpec((H, c), lambda i: (0, 0)),
            pl.BlockSpec((1, c), lambda i: (0, 0)),
            pl.BlockSpec((c, bn), lambda i: (0, 0)),
        ],
        out_specs=pl.BlockSpec((_BRM, bn), lambda i: (i, 0)),
        out_shape=jax.ShapeDtypeStruct((NROWS, bn), jnp.float32),
    )(y, aggs, aggs, b1s, w2, b2, w1n)


def _tc_final(y, aggs, b1s, w2, b2):
    c = w2.shape[1]
    nb = N // _BR

    def body(y_ref, a0_ref, a1_ref, b1_ref, w2_ref, b2_ref, o_ref):
        i = pl.program_id(0)
        u = jnp.maximum(y_ref[...] + a0_ref[0] + a1_ref[0] + b1_ref[...], 0.0)
        h = jnp.maximum(
            jnp.dot(u, w2_ref[...], preferred_element_type=jnp.float32)
            + b2_ref[...], 0.0)
        part = jnp.sum(h, axis=0, keepdims=True)

        @pl.when(i == 0)
        def _():
            o_ref[...] = part

        @pl.when(i > 0)
        def _():
            o_ref[...] += part

        @pl.when(i == nb - 1)
        def _():
            p = o_ref[...]
            m = jnp.max(p, axis=1, keepdims=True)
            e = jnp.exp(p - m)
            o_ref[...] = e / jnp.sum(e, axis=1, keepdims=True)

    return pl.pallas_call(
        body,
        grid=(nb,),
        in_specs=[
            pl.BlockSpec((_BR, H), lambda i: (i, 0)),
            pl.BlockSpec((1, _BR, H), lambda i: (0, i, 0)),
            pl.BlockSpec((1, _BR, H), lambda i: (1, i, 0)),
            pl.BlockSpec((1, H), lambda i: (0, 0)),
            pl.BlockSpec((H, c), lambda i: (0, 0)),
            pl.BlockSpec((1, c), lambda i: (0, 0)),
        ],
        out_specs=pl.BlockSpec((1, c), lambda i: (0, 0)),
        out_shape=jax.ShapeDtypeStruct((1, c), jnp.float32),
    )(y, aggs, aggs, b1s, w2, b2)


def kernel(x, edge_index, params):
    folded = []
    for p in params:
        s = p["g1"] / jnp.sqrt(1.0 + BN_EPS)
        w1s = p["W1"] * s[None, :]
        b1s = (p["b1"] * s + p["be1"]).reshape(1, -1)
        folded.append((w1s, b1s, p["W2"], p["b2"].reshape(1, -1)))

    pad = E_PAD - E
    src = jnp.concatenate(
        [edge_index[0], jnp.zeros((pad,), jnp.int32)]).reshape(NT, K, CHUNK)
    dst = jnp.concatenate(
        [edge_index[1], jnp.full((pad,), N, jnp.int32)]).reshape(NT, K, CHUNK)
    zeros = jnp.zeros((NROWS, H), jnp.float32)

    x = jnp.concatenate(
        [x, jnp.zeros((NROWS - N, x.shape[1]), jnp.float32)], axis=0)
    y = _tc_matmul(x, folded[0][0])
    for l in range(5):
        aggs = _sc_segment_sum(y, src, dst, zeros)
        w1s, b1s, w2, b2 = folded[l]
        if l < 4:
            y = _tc_mid(y, aggs, b1s, w2, b2, folded[l + 1][0])
        else:
            out = _tc_final(y, aggs, b1s, w2, b2)
    return out


# speedup vs baseline: 1.2716x; 1.1094x over previous
import functools

import jax
import jax.numpy as jnp
from jax import lax
from jax.experimental import pallas as pl
from jax.experimental.pallas import tpu as pltpu
from jax.experimental.pallas import tpu_sc as plsc

N = 10000
E = 320000
H = 32
BN_EPS = 1e-5

NC = 2
NS = 16
NT = NC * NS
CHUNK = 128
G = 8
K = 80
E_PAD = NT * K * CHUNK
ROWS_PER_TILE = 632
NROWS = NS * ROWS_PER_TILE


def _sc_segment_sum(y, src3, dst3, zeros):
    mesh = plsc.VectorSubcoreMesh(core_axis_name="c", subcore_axis_name="s")

    @functools.partial(
        pl.kernel,
        out_type=jax.ShapeDtypeStruct((NC, NROWS, 128), jnp.float32),
        mesh=mesh,
        compiler_params=pltpu.CompilerParams(use_tc_tiling_on_sc=False, skip_device_barrier=True),
        scratch_types=[
            pltpu.VMEM((K, CHUNK), jnp.int32),
            pltpu.VMEM((K, CHUNK), jnp.int32),
            pltpu.VMEM((2 * G, CHUNK, H), jnp.float32),
            pltpu.VMEM_SHARED((NROWS, H), jnp.float32),
            pltpu.VMEM_SHARED((NROWS, H), jnp.float32),
            pltpu.SemaphoreType.DMA,
            pltpu.SemaphoreType.DMA,
        ],
    )
    def k(y_hbm, src_hbm, dst_hbm, zeros_hbm, out_hbm, idx_s, idx_d, rows,
          y_sh, agg_sh, gsem, ssem):
        cid = lax.axis_index("c")
        sid = lax.axis_index("s")
        wid = sid * NC + cid
        row0 = sid * ROWS_PER_TILE
        pltpu.sync_copy(src_hbm.at[wid], idx_s)
        pltpu.sync_copy(dst_hbm.at[wid], idx_d)
        pltpu.sync_copy(y_hbm.at[pl.ds(row0, ROWS_PER_TILE)],
                        y_sh.at[pl.ds(row0, ROWS_PER_TILE)])
        pltpu.sync_copy(zeros_hbm.at[pl.ds(row0, ROWS_PER_TILE)],
                        agg_sh.at[pl.ds(row0, ROWS_PER_TILE)])
        plsc.subcore_barrier()

        def fire(base, half):
            return [pltpu.async_copy(y_sh.at[idx_s.at[base + b]],
                                     rows.at[half * G + b], gsem)
                    for b in range(G)]

        def scatter(base, half, gds):
            sds = []
            for b in range(G):
                gds[b].wait()
                sds.append(pltpu.async_copy(rows.at[half * G + b],
                                            agg_sh.at[idx_d.at[base + b]],
                                            ssem, add=True))
            return sds

        def body(t, carry):
            base = t * (2 * G)
            gA = fire(base, 0)
            sA = scatter(base, 0, gA)
            gB = fire(base + G, 1)
            for d in sA:
                d.wait()
            sB = scatter(base + G, 1, gB)
            for d in sB:
                d.wait()
            return carry

        lax.fori_loop(0, K // (2 * G), body, 0)
        plsc.subcore_barrier()
        pltpu.sync_copy(agg_sh.at[pl.ds(row0, ROWS_PER_TILE)],
                        out_hbm.at[cid, pl.ds(row0, ROWS_PER_TILE), pl.ds(0, H)])

    return k(y, src3, dst3, zeros)


_BR = 1000
_BRM = 2528


def _tc_matmul(x, w):
    a, b = x.shape[1], w.shape[1]

    def body(x_ref, w_ref, o_ref):
        o_ref[...] = jnp.dot(x_ref[...], w_ref[...],
                             preferred_element_type=jnp.float32)

    return pl.pallas_call(
        body,
        grid=(NROWS // _BR,),
        in_specs=[pl.BlockSpec((_BR, a), lambda i: (i, 0)),
                  pl.BlockSpec((a, b), lambda i: (0, 0))],
        out_specs=pl.BlockSpec((_BR, b), lambda i: (i, 0)),
        out_shape=jax.ShapeDtypeStruct((NROWS, b), jnp.float32),
    )(x, w)


def _tc_mid(y, aggs, b1s, w2, b2, w1n):
    c = w2.shape[1]
    bn = w1n.shape[1]

    def body(y_ref, a0_ref, a1_ref, b1_ref, w2_ref, b2_ref, w1_ref, o_ref):
        u = jnp.maximum(
            y_ref[...] + a0_ref[0][:, :H] + a1_ref[0][:, :H] + b1_ref[...],
            0.0)
        h = jnp.maximum(
            jnp.dot(u, w2_ref[...], preferred_element_type=jnp.float32)
            + b2_ref[...], 0.0)
        o_ref[...] = jnp.dot(h, w1_ref[...], preferred_element_type=jnp.float32)

    return pl.pallas_call(
        body,
        grid=(NROWS // _BRM,),
        in_specs=[
            pl.BlockSpec((_BRM, H), lambda i: (i, 0)),
            pl.BlockSpec((1, _BRM, 128), lambda i: (0, i, 0)),
            pl.BlockSpec((1, _BRM, 128), lambda i: (1, i, 0)),
            pl.BlockSpec((1, H), lambda i: (0, 0)),
            pl.BlockSpec((H, c), lambda i: (0, 0)),
            pl.BlockSpec((1, c), lambda i: (0, 0)),
            pl.BlockSpec((c, bn), lambda i: (0, 0)),
        ],
        out_specs=pl.BlockSpec((_BRM, bn), lambda i: (i, 0)),
        out_shape=jax.ShapeDtypeStruct((NROWS, bn), jnp.float32),
    )(y, aggs, aggs, b1s, w2, b2, w1n)


def _tc_final(y, aggs, b1s, w2, b2):
    c = w2.shape[1]
    nb = N // _BR

    def body(y_ref, a0_ref, a1_ref, b1_ref, w2_ref, b2_ref, o_ref):
        i = pl.program_id(0)
        u = jnp.maximum(
            y_ref[...] + a0_ref[0][:, :H] + a1_ref[0][:, :H] + b1_ref[...],
            0.0)
        h = jnp.maximum(
            jnp.dot(u, w2_ref[...], preferred_element_type=jnp.float32)
            + b2_ref[...], 0.0)
        part = jnp.sum(h, axis=0, keepdims=True)

        @pl.when(i == 0)
        def _():
            o_ref[...] = part

        @pl.when(i > 0)
        def _():
            o_ref[...] += part

        @pl.when(i == nb - 1)
        def _():
            p = o_ref[...]
            m = jnp.max(p, axis=1, keepdims=True)
            e = jnp.exp(p - m)
            o_ref[...] = e / jnp.sum(e, axis=1, keepdims=True)

    return pl.pallas_call(
        body,
        grid=(nb,),
        in_specs=[
            pl.BlockSpec((_BR, H), lambda i: (i, 0)),
            pl.BlockSpec((1, _BR, 128), lambda i: (0, i, 0)),
            pl.BlockSpec((1, _BR, 128), lambda i: (1, i, 0)),
            pl.BlockSpec((1, H), lambda i: (0, 0)),
            pl.BlockSpec((H, c), lambda i: (0, 0)),
            pl.BlockSpec((1, c), lambda i: (0, 0)),
        ],
        out_specs=pl.BlockSpec((1, c), lambda i: (0, 0)),
        out_shape=jax.ShapeDtypeStruct((1, c), jnp.float32),
    )(y, aggs, aggs, b1s, w2, b2)


def kernel(x, edge_index, params):
    folded = []
    for p in params:
        s = p["g1"] / jnp.sqrt(1.0 + BN_EPS)
        w1s = p["W1"] * s[None, :]
        b1s = (p["b1"] * s + p["be1"]).reshape(1, -1)
        folded.append((w1s, b1s, p["W2"], p["b2"].reshape(1, -1)))

    pad = E_PAD - E
    src = jnp.concatenate(
        [edge_index[0], jnp.zeros((pad,), jnp.int32)]).reshape(NT, K, CHUNK)
    dst = jnp.concatenate(
        [edge_index[1], jnp.full((pad,), N, jnp.int32)]).reshape(NT, K, CHUNK)
    zeros = jnp.zeros((NROWS, H), jnp.float32)

    x = jnp.concatenate(
        [x, jnp.zeros((NROWS - N, x.shape[1]), jnp.float32)], axis=0)
    y = _tc_matmul(x, folded[0][0])
    for l in range(5):
        aggs = _sc_segment_sum(y, src, dst, zeros)
        w1s, b1s, w2, b2 = folded[l]
        if l < 4:
            y = _tc_mid(y, aggs, b1s, w2, b2, folded[l + 1][0])
        else:
            out = _tc_final(y, aggs, b1s, w2, b2)
    return out


# speedup vs baseline: 1.3281x; 1.0444x over previous
import functools

import jax
import jax.numpy as jnp
from jax import lax
from jax.experimental import pallas as pl
from jax.experimental.pallas import tpu as pltpu
from jax.experimental.pallas import tpu_sc as plsc

N = 10000
E = 320000
H = 32
BN_EPS = 1e-5

NC = 2
NS = 16
NT = NC * NS
CHUNK = 128
G = 8
K = 80
E_PAD = NT * K * CHUNK
ROWS_PER_TILE = 632
NROWS = NS * ROWS_PER_TILE


def _sc_segment_sum(y, src3, dst3, zeros):
    mesh = plsc.VectorSubcoreMesh(core_axis_name="c", subcore_axis_name="s")

    @functools.partial(
        pl.kernel,
        out_type=jax.ShapeDtypeStruct((NC, NROWS, 128), jnp.float32),
        mesh=mesh,
        compiler_params=pltpu.CompilerParams(use_tc_tiling_on_sc=False, skip_device_barrier=True),
        scratch_types=[
            pltpu.VMEM((K, CHUNK), jnp.int32),
            pltpu.VMEM((K, CHUNK), jnp.int32),
            pltpu.VMEM((2 * G, CHUNK, H), jnp.float32),
            pltpu.VMEM_SHARED((NROWS, H), jnp.float32),
            pltpu.VMEM_SHARED((NROWS, H), jnp.float32),
            pltpu.SemaphoreType.DMA,
            pltpu.SemaphoreType.DMA,
        ],
    )
    def k(y_hbm, src_hbm, dst_hbm, zeros_hbm, out_hbm, idx_s, idx_d, rows,
          y_sh, agg_sh, gsem, ssem):
        cid = lax.axis_index("c")
        sid = lax.axis_index("s")
        wid = sid * NC + cid
        row0 = sid * ROWS_PER_TILE
        pltpu.sync_copy(src_hbm.at[wid], idx_s)
        pltpu.sync_copy(dst_hbm.at[wid], idx_d)
        pltpu.sync_copy(y_hbm.at[pl.ds(row0, ROWS_PER_TILE), pl.ds(0, H)],
                        y_sh.at[pl.ds(row0, ROWS_PER_TILE)])
        pltpu.sync_copy(zeros_hbm.at[pl.ds(row0, ROWS_PER_TILE)],
                        agg_sh.at[pl.ds(row0, ROWS_PER_TILE)])
        plsc.subcore_barrier()

        def fire(base, half):
            return [pltpu.async_copy(y_sh.at[idx_s.at[base + b]],
                                     rows.at[half * G + b], gsem)
                    for b in range(G)]

        def scatter(base, half, gds):
            sds = []
            for b in range(G):
                gds[b].wait()
                sds.append(pltpu.async_copy(rows.at[half * G + b],
                                            agg_sh.at[idx_d.at[base + b]],
                                            ssem, add=True))
            return sds

        def body(t, carry):
            base = t * (2 * G)
            gA = fire(base, 0)
            sA = scatter(base, 0, gA)
            gB = fire(base + G, 1)
            for d in sA:
                d.wait()
            sB = scatter(base + G, 1, gB)
            for d in sB:
                d.wait()
            return carry

        lax.fori_loop(0, K // (2 * G), body, 0)
        plsc.subcore_barrier()
        pltpu.sync_copy(agg_sh.at[pl.ds(row0, ROWS_PER_TILE)],
                        out_hbm.at[cid, pl.ds(row0, ROWS_PER_TILE), pl.ds(0, H)])

    return k(y, src3, dst3, zeros)


_BR = 1000
_BRM = 2528


def _tc_matmul(x, w):
    a, b = x.shape[1], w.shape[1]

    def body(x_ref, w_ref, o_ref):
        o_ref[...] = jnp.dot(x_ref[...], w_ref[...],
                             preferred_element_type=jnp.float32)

    return pl.pallas_call(
        body,
        grid=(NROWS // _BR,),
        in_specs=[pl.BlockSpec((_BR, a), lambda i: (i, 0)),
                  pl.BlockSpec((a, b), lambda i: (0, 0))],
        out_specs=pl.BlockSpec((_BR, b), lambda i: (i, 0)),
        out_shape=jax.ShapeDtypeStruct((NROWS, b), jnp.float32),
    )(x, w)


def _tc_mid(y, aggs, b1s, w2, b2, w1n):
    c = w2.shape[1]
    bn = w1n.shape[1]

    def body(y_ref, a0_ref, a1_ref, b1_ref, w2_ref, b2_ref, w1_ref, o_ref):
        u = jnp.maximum(
            y_ref[:, :H] + a0_ref[0][:, :H] + a1_ref[0][:, :H] + b1_ref[...],
            0.0)
        h = jnp.maximum(
            jnp.dot(u, w2_ref[...], preferred_element_type=jnp.float32)
            + b2_ref[...], 0.0)
        o_ref[...] = jnp.dot(h, w1_ref[...], preferred_element_type=jnp.float32)

    return pl.pallas_call(
        body,
        grid=(NROWS // _BRM,),
        in_specs=[
            pl.BlockSpec((_BRM, 128), lambda i: (i, 0)),
            pl.BlockSpec((1, _BRM, 128), lambda i: (0, i, 0)),
            pl.BlockSpec((1, _BRM, 128), lambda i: (1, i, 0)),
            pl.BlockSpec((1, H), lambda i: (0, 0)),
            pl.BlockSpec((H, c), lambda i: (0, 0)),
            pl.BlockSpec((1, c), lambda i: (0, 0)),
            pl.BlockSpec((c, bn), lambda i: (0, 0)),
        ],
        out_specs=pl.BlockSpec((_BRM, bn), lambda i: (i, 0)),
        out_shape=jax.ShapeDtypeStruct((NROWS, bn), jnp.float32),
    )(y, aggs, aggs, b1s, w2, b2, w1n)


def _tc_final(y, aggs, b1s, w2, b2):
    c = w2.shape[1]
    nb = N // _BR

    def body(y_ref, a0_ref, a1_ref, b1_ref, w2_ref, b2_ref, o_ref):
        i = pl.program_id(0)
        u = jnp.maximum(
            y_ref[:, :H] + a0_ref[0][:, :H] + a1_ref[0][:, :H] + b1_ref[...],
            0.0)
        h = jnp.maximum(
            jnp.dot(u, w2_ref[...], preferred_element_type=jnp.float32)
            + b2_ref[...], 0.0)
        part = jnp.sum(h, axis=0, keepdims=True)

        @pl.when(i == 0)
        def _():
            o_ref[...] = part

        @pl.when(i > 0)
        def _():
            o_ref[...] += part

        @pl.when(i == nb - 1)
        def _():
            p = o_ref[...]
            m = jnp.max(p, axis=1, keepdims=True)
            e = jnp.exp(p - m)
            o_ref[...] = e / jnp.sum(e, axis=1, keepdims=True)

    return pl.pallas_call(
        body,
        grid=(nb,),
        in_specs=[
            pl.BlockSpec((_BR, 128), lambda i: (i, 0)),
            pl.BlockSpec((1, _BR, 128), lambda i: (0, i, 0)),
            pl.BlockSpec((1, _BR, 128), lambda i: (1, i, 0)),
            pl.BlockSpec((1, H), lambda i: (0, 0)),
            pl.BlockSpec((H, c), lambda i: (0, 0)),
            pl.BlockSpec((1, c), lambda i: (0, 0)),
        ],
        out_specs=pl.BlockSpec((1, c), lambda i: (0, 0)),
        out_shape=jax.ShapeDtypeStruct((1, c), jnp.float32),
    )(y, aggs, aggs, b1s, w2, b2)


def kernel(x, edge_index, params):
    folded = []
    for p in params:
        s = p["g1"] / jnp.sqrt(1.0 + BN_EPS)
        w1s = p["W1"] * s[None, :]
        w1s = jnp.pad(w1s, ((0, 0), (0, 128 - w1s.shape[1])))
        b1s = (p["b1"] * s + p["be1"]).reshape(1, -1)
        folded.append((w1s, b1s, p["W2"], p["b2"].reshape(1, -1)))

    pad = E_PAD - E
    src = jnp.concatenate(
        [edge_index[0], jnp.zeros((pad,), jnp.int32)]).reshape(NT, K, CHUNK)
    dst = jnp.concatenate(
        [edge_index[1], jnp.full((pad,), N, jnp.int32)]).reshape(NT, K, CHUNK)
    zeros = jnp.zeros((NROWS, H), jnp.float32)

    x = jnp.concatenate(
        [x, jnp.zeros((NROWS - N, x.shape[1]), jnp.float32)], axis=0)
    y = _tc_matmul(x, folded[0][0])
    for l in range(5):
        aggs = _sc_segment_sum(y, src, dst, zeros)
        w1s, b1s, w2, b2 = folded[l]
        if l < 4:
            y = _tc_mid(y, aggs, b1s, w2, b2, folded[l + 1][0])
        else:
            out = _tc_final(y, aggs, b1s, w2, b2)
    return out
